# Initial kernel scaffold; baseline (speedup 1.0000x reference)
import jax
import jax.numpy as jnp
from jax.experimental import pallas as pl


def kernel(x, W_enc, b_enc, W_dec, b_dec):
    raise NotImplementedError("write your pallas kernel here")


# speedup vs baseline: 1.8003x; 1.8003x over previous
import functools

import jax
import jax.numpy as jnp
from jax import lax
from jax.experimental import pallas as pl
from jax.experimental.pallas import tpu as pltpu
from jax.experimental.pallas import tpu_sc as plsc

_B = 256
_DIN = 1024
_S = 32
_V = 512
_EPS = 1e-20

_NW = 32
_RPW = _B // _NW


def _tc_body(x_ref, w_ref, b_ref, g_ref, z_ref, h_ref, k_ref, gid_ref):
    s = pl.program_id(0)
    z = jnp.dot(x_ref[...], w_ref[...], preferred_element_type=jnp.float32)
    z = z + b_ref[...]
    z_ref[...] = z
    gum = z + g_ref[...]
    m = jnp.max(gum, axis=1, keepdims=True)
    e = jnp.exp(gum - m)
    y = e / jnp.sum(e, axis=1, keepdims=True)
    m2 = jnp.max(y, axis=1, keepdims=True)
    eq = y == m2
    iota = lax.broadcasted_iota(jnp.int32, (_B, _V), 1)
    k = jnp.min(jnp.where(eq, iota, _V), axis=1, keepdims=True)
    h_ref[...] = jnp.where(iota == k, 1.0, 0.0).astype(jnp.float32)
    col = lax.broadcasted_iota(jnp.int32, (_B, _S), 1)
    k_ref[...] = jnp.where(col == s, k, k_ref[...])
    gid_ref[...] = jnp.where(col == s, k + _V * s, gid_ref[...])


def _tc_encode(x, W_enc, b_enc2, g2):
    return pl.pallas_call(
        _tc_body,
        grid=(_S,),
        in_specs=[
            pl.BlockSpec((_B, _DIN), lambda s: (0, 0)),
            pl.BlockSpec((_DIN, _V), lambda s: (0, s)),
            pl.BlockSpec((1, _V), lambda s: (0, s)),
            pl.BlockSpec((_B, _V), lambda s: (0, s)),
        ],
        out_specs=[
            pl.BlockSpec((_B, _V), lambda s: (0, s)),
            pl.BlockSpec((_B, _V), lambda s: (0, s)),
            pl.BlockSpec((_B, _S), lambda s: (0, 0)),
            pl.BlockSpec((_B, _S), lambda s: (0, 0)),
        ],
        out_shape=[
            jax.ShapeDtypeStruct((_B, _S * _V), jnp.float32),
            jax.ShapeDtypeStruct((_B, _S * _V), jnp.float32),
            jax.ShapeDtypeStruct((_B, _S), jnp.int32),
            jax.ShapeDtypeStruct((_B, _S), jnp.int32),
        ],
    )(x, W_enc, b_enc2, g2)


def _sc_decode_body(gid_hbm, wdec_hbm, bdec_hbm, out_hbm,
                    idx_v, rows_v, bdec_v, obuf_v, sem):
    cid = lax.axis_index("c")
    sid = lax.axis_index("s")
    wid = sid * 2 + cid
    base_b = wid * _RPW
    pltpu.sync_copy(bdec_hbm, bdec_v)

    def row_body(j, carry):
        b = base_b + j
        pltpu.sync_copy(gid_hbm.at[pl.ds(b * _S, _S)], idx_v)
        pltpu.async_copy(wdec_hbm.at[idx_v], rows_v, sem).wait()

        def col_body(c, carry2):
            o = c * 16
            a0 = rows_v[0, pl.ds(o, 16)]
            a1 = rows_v[1, pl.ds(o, 16)]
            a2 = rows_v[2, pl.ds(o, 16)]
            a3 = rows_v[3, pl.ds(o, 16)]
            for t in range(4, _S, 4):
                a0 = a0 + rows_v[t + 0, pl.ds(o, 16)]
                a1 = a1 + rows_v[t + 1, pl.ds(o, 16)]
                a2 = a2 + rows_v[t + 2, pl.ds(o, 16)]
                a3 = a3 + rows_v[t + 3, pl.ds(o, 16)]
            acc = bdec_v[pl.ds(o, 16)] + ((a0 + a1) + (a2 + a3))
            obuf_v[j, pl.ds(o, 16)] = acc
            return carry2

        lax.fori_loop(0, _DIN // 16, col_body, 0, unroll=False)
        return carry

    lax.fori_loop(0, _RPW, row_body, 0, unroll=False)
    pltpu.sync_copy(obuf_v, out_hbm.at[pl.ds(base_b, _RPW)])


def _sc_decode(gid_flat, W_dec, b_dec):
    mesh = plsc.VectorSubcoreMesh(core_axis_name="c", subcore_axis_name="s")
    f = pl.kernel(
        _sc_decode_body,
        out_type=jax.ShapeDtypeStruct((_B, _DIN), jnp.float32),
        mesh=mesh,
        scratch_types=[
            pltpu.VMEM((_S,), jnp.int32),
            pltpu.VMEM((_S, _DIN), jnp.float32),
            pltpu.VMEM((_DIN,), jnp.float32),
            pltpu.VMEM((_RPW, _DIN), jnp.float32),
            pltpu.SemaphoreType.DMA,
        ],
    )
    return f(gid_flat, W_dec, b_dec)


def kernel(x, W_enc, b_enc, W_dec, b_dec):
    u = jax.random.uniform(jax.random.key(42), (_B, 1, _S, _V), dtype=jnp.float32)
    g = -jnp.log(-jnp.log(u + _EPS) + _EPS)
    g2 = g.reshape(_B, _S * _V)

    z, hard, kmat, gid = _tc_encode(x, W_enc, b_enc.reshape(1, _S * _V), g2)
    x_hat = _sc_decode(gid.reshape(_B * _S), W_dec, b_dec)
    return (z, kmat.reshape(_B, 1, _S), hard, x_hat)
